# baseline (speedup 1.0000x reference)
import functools

import jax
import jax.numpy as jnp
from jax import lax
from jax.experimental import pallas as pl
from jax.experimental.pallas import tpu as pltpu
from jax.experimental.pallas import tpu_sc as plsc

VOCAB = 1000000
L = 200
D = 32
B = 4096
N = B * L

NC, NS = 2, 16
NW = NC * NS

TG = VOCAB // 128
TG_MAIN = (TG // (2 * NW)) * (2 * NW)
K1_PAIRS = TG_MAIN // (2 * NW)
WIDE = VOCAB // 4


def _iota16():
    return lax.iota(jnp.int32, 16)


@functools.partial(
    pl.kernel,
    mesh=plsc.VectorSubcoreMesh(core_axis_name="c", subcore_axis_name="s"),
    out_type=jax.ShapeDtypeStruct((WIDE, 128), jnp.float32),
    scratch_types=[
        pltpu.VMEM((2, 32, 128), jnp.float32),
        pltpu.VMEM((2, 32, 128), jnp.float32),
        pltpu.SemaphoreType.DMA,
        pltpu.SemaphoreType.DMA,
        pltpu.SemaphoreType.DMA,
        pltpu.SemaphoreType.DMA,
    ],
    compiler_params=pltpu.CompilerParams(needs_layout_passes=False),
)
def _detr_kernel(tblT_hbm, wide_hbm, stage_v, line_v, gsem0, gsem1, ssem0, ssem1):
    wid = lax.axis_index("s") * NC + lax.axis_index("c")
    iota = _iota16()

    def fire_load(g, buf, sem):
        pltpu.async_copy(tblT_hbm.at[:, pl.ds(g * 128, 128)], stage_v.at[buf], sem)

    def drain_load(buf, sem):
        pltpu.make_async_copy(
            tblT_hbm.at[:, pl.ds(0, 128)], stage_v.at[buf], sem).wait()

    def drain_store(buf, sem):
        pltpu.make_async_copy(
            line_v.at[buf], wide_hbm.at[pl.ds(0, 32)], sem).wait()

    def transpose_group(buf):
        def j_body(j, carry):
            for h in range(2):
                rows = iota + (h * 16)
                for k in range(4):
                    col = jnp.full((16,), 4 * j + k, jnp.int32)
                    v = plsc.load_gather(stage_v.at[buf], [rows, col])
                    line_v[buf, j, pl.ds(k * 32 + h * 16, 16)] = v
            return carry

        lax.fori_loop(0, 32, j_body, 0)

    def fire_store(g, buf, sem):
        pltpu.async_copy(line_v.at[buf], wide_hbm.at[pl.ds(g * 32, 32)], sem)

    fire_load(wid, 0, gsem0)

    def pair_body(i, carry):
        ga = (2 * i) * NW + wid
        gb = (2 * i + 1) * NW + wid
        drain_load(0, gsem0)
        fire_load(gb, 1, gsem1)

        @pl.when(i > 0)
        def _():
            drain_store(0, ssem0)

        transpose_group(0)
        fire_store(ga, 0, ssem0)
        drain_load(1, gsem1)

        @pl.when(i + 1 < K1_PAIRS)
        def _():
            fire_load((2 * i + 2) * NW + wid, 0, gsem0)

        @pl.when(i > 0)
        def _():
            drain_store(1, ssem1)

        transpose_group(1)
        fire_store(gb, 1, ssem1)
        return carry

    lax.fori_loop(0, K1_PAIRS, pair_body, 0)
    drain_store(0, ssem0)
    drain_store(1, ssem1)

    @pl.when(wid < TG - TG_MAIN)
    def _():
        g = TG_MAIN + wid
        fire_load(g, 0, gsem0)
        drain_load(0, gsem0)
        transpose_group(0)
        fire_store(g, 0, ssem0)
        drain_store(0, ssem0)

    @pl.when(wid == 4)
    def _():
        tail_col = wid * (TG * 128 // 4)
        pltpu.async_copy(
            tblT_hbm.at[:, pl.ds(tail_col, 128)], stage_v.at[1], gsem1)
        drain_load(1, gsem1)

        def j_body(j, carry):
            for h in range(2):
                rows = iota + (h * 16)
                for k in range(4):
                    col = jnp.full((16,), 4 * j + k, jnp.int32)
                    v = plsc.load_gather(stage_v.at[1], [rows, col])
                    line_v[1, j, pl.ds(k * 32 + h * 16, 16)] = v
            return carry

        lax.fori_loop(0, 16, j_body, 0)
        pltpu.async_copy(
            line_v.at[1, pl.ds(0, 16)],
            wide_hbm.at[pl.ds(TG * 32, 16)], ssem1)
        pltpu.make_async_copy(
            line_v.at[1, pl.ds(0, 16)],
            wide_hbm.at[pl.ds(0, 16)], ssem1).wait()


LC = 5
NCHUNK = L // LC
K2_PAIRS = NCHUNK // 2


@functools.partial(
    pl.kernel,
    mesh=plsc.VectorSubcoreMesh(core_axis_name="c", subcore_axis_name="s"),
    out_type=jax.ShapeDtypeStruct((L, D // 8, B // 128, 8, 128), jnp.float32),
    scratch_types=[
        pltpu.VMEM((L, 128), jnp.int32),
        pltpu.VMEM((L * D,), jnp.float32),
        pltpu.VMEM((2, LC, 128, D), jnp.float32),
        pltpu.VMEM((2, LC, D // 8, 8, 128), jnp.float32),
        pltpu.SemaphoreType.DMA,
        pltpu.SemaphoreType.DMA,
        pltpu.SemaphoreType.DMA,
        pltpu.SemaphoreType.DMA,
    ],
    compiler_params=pltpu.CompilerParams(
        use_tc_tiling_on_sc=False, needs_layout_passes=False),
)
def _gather_kernel(xT_hbm, tok_hbm, pos_hbm, out_hbm, idx_v, pos_v, rows_v,
                   st_v, gsem0, gsem1, ssem0, ssem1):
    wid = lax.axis_index("s") * NC + lax.axis_index("c")
    gsem = (gsem0, gsem1)
    ssem = (ssem0, ssem1)
    iota = _iota16()

    pltpu.sync_copy(xT_hbm.at[:, pl.ds(wid * 128, 128)], idx_v)
    pltpu.sync_copy(pos_hbm, pos_v)

    def fire_gathers(c, buf):
        l0 = c * LC
        for li in range(LC):
            pltpu.async_copy(
                tok_hbm.at[idx_v.at[l0 + li]], rows_v.at[buf, li], gsem[buf])

    def drain_gathers(buf):
        pltpu.make_async_copy(
            tok_hbm.at[idx_v.at[0]], rows_v.at[buf, 0], gsem[buf]).wait()

    def drain_gathers_all(buf):
        for _ in range(LC):
            drain_gathers(buf)

    def drain_store(buf):
        pltpu.make_async_copy(
            st_v.at[buf], out_hbm.at[pl.ds(0, LC), :, 0], ssem[buf]).wait()

    def transpose_chunk(c, buf):
        l0 = c * LC

        def t_body(t, carry):
            li = t // (D // 8)
            dt = t % (D // 8)
            for d in range(8):
                dc = dt * 8 + d
                pidx = jnp.full((16,), (l0 + li) * D + dc, jnp.int32)
                pv = plsc.load_gather(pos_v, [pidx])
                col = jnp.full((16,), dc, jnp.int32)
                for bg in range(8):
                    rows = plsc.load_gather(
                        rows_v.at[buf, li], [iota + bg * 16, col])
                    st_v[buf, li, dt, d, pl.ds(bg * 16, 16)] = rows + pv
            return carry

        lax.fori_loop(0, LC * (D // 8), t_body, 0)

    def fire_store(c, buf):
        l0 = c * LC
        pltpu.async_copy(st_v.at[buf], out_hbm.at[pl.ds(l0, LC), :, wid], ssem[buf])

    fire_gathers(0, 0)

    def pair_body(i, carry):
        ca = 2 * i
        cb = 2 * i + 1
        fire_gathers(cb, 1)
        drain_gathers_all(0)

        @pl.when(i > 0)
        def _():
            drain_store(0)

        transpose_chunk(ca, 0)
        fire_store(ca, 0)

        @pl.when(i + 1 < K2_PAIRS)
        def _():
            fire_gathers(ca + 2, 0)

        drain_gathers_all(1)

        @pl.when(i > 0)
        def _():
            drain_store(1)

        transpose_chunk(cb, 1)
        fire_store(cb, 1)
        return carry

    lax.fori_loop(0, K2_PAIRS, pair_body, 0)
    drain_store(0)
    drain_store(1)


def kernel(x, token_table, pos_table):
    tblT = token_table.T
    wide = _detr_kernel(tblT)
    tok_lin = wide.reshape(VOCAB, D)
    xT = x.astype(jnp.int32).T
    pos_f = pos_table.reshape(L * D)
    out5d = _gather_kernel(xT, tok_lin, pos_f)
    return out5d.transpose((2, 4, 0, 1, 3)).reshape(B, L, D)


# speedup vs baseline: 1.2231x; 1.2231x over previous
import functools

import jax
import jax.numpy as jnp
from jax import lax
from jax.experimental import pallas as pl
from jax.experimental.pallas import tpu as pltpu
from jax.experimental.pallas import tpu_sc as plsc

VOCAB = 1000000
L = 200
D = 32
B = 4096
N = B * L

NC, NS = 2, 16
NW = NC * NS

TG = VOCAB // 128
TG_MAIN = (TG // (2 * NW)) * (2 * NW)
K1_PAIRS = TG_MAIN // (2 * NW)


def _iota16():
    return lax.iota(jnp.int32, 16)


@functools.partial(
    pl.kernel,
    mesh=plsc.VectorSubcoreMesh(core_axis_name="c", subcore_axis_name="s"),
    out_type=jax.ShapeDtypeStruct((VOCAB * D,), jnp.float32),
    scratch_types=[
        pltpu.VMEM((32, 128), jnp.float32),
        pltpu.VMEM((32, 128), jnp.float32),
        pltpu.VMEM((4096,), jnp.float32),
        pltpu.VMEM((4096,), jnp.float32),
        pltpu.SemaphoreType.DMA,
        pltpu.SemaphoreType.DMA,
        pltpu.SemaphoreType.DMA,
        pltpu.SemaphoreType.DMA,
    ],
    compiler_params=pltpu.CompilerParams(needs_layout_passes=False),
)
def _detr_kernel(tblT_hbm, out_hbm, stage0, stage1, line0, line1,
                 gsem0, gsem1, ssem0, ssem1):
    stage_v = (stage0, stage1)
    line_v = (line0, line1)
    wid = lax.axis_index("s") * NC + lax.axis_index("c")
    iota = _iota16()
    patq = ((iota >> 2) << 7) + ((iota & 3) << 5)

    def fire_load(g, buf, sem):
        pltpu.async_copy(tblT_hbm.at[:, pl.ds(g * 128, 128)], stage_v[buf], sem)

    def drain_load(buf, sem):
        pltpu.make_async_copy(
            tblT_hbm.at[:, pl.ds(0, 128)], stage_v[buf], sem).wait()

    def drain_store(buf, sem):
        pltpu.make_async_copy(
            line_v[buf], out_hbm.at[pl.ds(0, 4096)], sem).wait()

    def transpose_group(buf, ngroups=8):
        def d_body(d, carry):
            for tg in range(ngroups):
                v = stage_v[buf][d, pl.ds(tg * 16, 16)]
                plsc.store_scatter(line_v[buf], [patq + (tg * 512 + d)], v)
            return carry

        lax.fori_loop(0, 32, d_body, 0)

    def fire_store(g, buf, sem):
        pltpu.async_copy(line_v[buf], out_hbm.at[pl.ds(g * 4096, 4096)], sem)

    fire_load(wid, 0, gsem0)

    def pair_body(i, carry):
        ga = (2 * i) * NW + wid
        gb = (2 * i + 1) * NW + wid
        drain_load(0, gsem0)
        fire_load(gb, 1, gsem1)

        @pl.when(i > 0)
        def _():
            drain_store(0, ssem0)

        transpose_group(0)
        fire_store(ga, 0, ssem0)
        drain_load(1, gsem1)

        @pl.when(i + 1 < K1_PAIRS)
        def _():
            fire_load((2 * i + 2) * NW + wid, 0, gsem0)

        @pl.when(i > 0)
        def _():
            drain_store(1, ssem1)

        transpose_group(1)
        fire_store(gb, 1, ssem1)
        return carry

    lax.fori_loop(0, K1_PAIRS, pair_body, 0)
    drain_store(0, ssem0)
    drain_store(1, ssem1)

    @pl.when(wid < TG - TG_MAIN)
    def _():
        g = TG_MAIN + wid
        fire_load(g, 0, gsem0)
        drain_load(0, gsem0)
        transpose_group(0)
        fire_store(g, 0, ssem0)
        drain_store(0, ssem0)

    @pl.when(wid == 4)
    def _():
        tail_col = wid * (TG * 128 // 4)
        pltpu.async_copy(
            tblT_hbm.at[:, pl.ds(tail_col, 128)], stage_v[1], gsem1)
        drain_load(1, gsem1)
        transpose_group(1, ngroups=4)
        pltpu.async_copy(
            line_v[1].at[pl.ds(0, 2048)],
            out_hbm.at[pl.ds(TG * 4096, 2048)], ssem1)
        pltpu.make_async_copy(
            line_v[1].at[pl.ds(0, 2048)],
            out_hbm.at[pl.ds(0, 2048)], ssem1).wait()


LC = 5
NCHUNK = L // LC
K2_PAIRS = NCHUNK // 2
CH_OUT = LC * D * 128


@functools.partial(
    pl.kernel,
    mesh=plsc.VectorSubcoreMesh(core_axis_name="c", subcore_axis_name="s"),
    out_type=jax.ShapeDtypeStruct((N * D,), jnp.float32),
    scratch_types=[
        pltpu.VMEM((L, 128), jnp.int32),
        pltpu.VMEM((L * D,), jnp.float32),
        pltpu.VMEM((LC, 128, D), jnp.float32),
        pltpu.VMEM((LC, 128, D), jnp.float32),
        pltpu.VMEM((CH_OUT,), jnp.float32),
        pltpu.VMEM((CH_OUT,), jnp.float32),
        pltpu.SemaphoreType.DMA,
        pltpu.SemaphoreType.DMA,
        pltpu.SemaphoreType.DMA,
        pltpu.SemaphoreType.DMA,
    ],
    compiler_params=pltpu.CompilerParams(
        use_tc_tiling_on_sc=False, needs_layout_passes=False),
)
def _gather_kernel(xT_hbm, tok_hbm, pos_hbm, out_hbm, idx_v, pos_v, rows0,
                   rows1, st0, st1, gsem0, gsem1, ssem0, ssem1):
    rows_v = (rows0, rows1)
    st_v = (st0, st1)
    wid = lax.axis_index("s") * NC + lax.axis_index("c")
    gsem = (gsem0, gsem1)
    ssem = (ssem0, ssem1)
    iota = _iota16()
    patd = ((iota >> 3) << 10) + ((iota & 7) << 7)

    pltpu.sync_copy(xT_hbm.at[:, pl.ds(wid * 128, 128)], idx_v)
    pltpu.sync_copy(pos_hbm, pos_v)

    def fire_gathers(c, buf):
        l0 = c * LC
        for li in range(LC):
            pltpu.async_copy(
                tok_hbm.at[idx_v.at[l0 + li]], rows_v[buf].at[li], gsem[buf])

    def drain_gathers_all(buf):
        for _ in range(LC):
            pltpu.make_async_copy(
                tok_hbm.at[idx_v.at[0]], rows_v[buf].at[0], gsem[buf]).wait()

    def drain_store(buf):
        pltpu.make_async_copy(
            st_v[buf], out_hbm.at[pl.ds(0, CH_OUT)], ssem[buf]).wait()

    def transpose_chunk(c, buf):
        l0 = c * LC
        pv = [[pos_v[pl.ds((l0 + li) * D + h * 16, 16)] for h in range(2)]
              for li in range(LC)]

        def b_body(b, carry):
            for li in range(LC):
                for h in range(2):
                    v = rows_v[buf][li, b, pl.ds(h * 16, 16)] + pv[li][h]
                    plsc.store_scatter(
                        st_v[buf],
                        [patd + (li * 4096 + h * 2048 + b)], v)
            return carry

        lax.fori_loop(0, 128, b_body, 0)

    def fire_store(c, buf):
        l0 = c * LC
        for li in range(LC):
            for dt in range(4):
                pltpu.async_copy(
                    st_v[buf].at[pl.ds((li * 4 + dt) * 1024, 1024)],
                    out_hbm.at[pl.ds(((l0 + li) * 4 + dt) * (B // 128 * 1024)
                                     + wid * 1024, 1024)],
                    ssem[buf])

    fire_gathers(0, 0)

    def pair_body(i, carry):
        ca = 2 * i
        cb = 2 * i + 1
        fire_gathers(cb, 1)
        drain_gathers_all(0)

        @pl.when(i > 0)
        def _():
            drain_store(0)

        transpose_chunk(ca, 0)
        fire_store(ca, 0)

        @pl.when(i + 1 < K2_PAIRS)
        def _():
            fire_gathers(ca + 2, 0)

        drain_gathers_all(1)

        @pl.when(i > 0)
        def _():
            drain_store(1)

        transpose_chunk(cb, 1)
        fire_store(cb, 1)
        return carry

    lax.fori_loop(0, K2_PAIRS, pair_body, 0)
    drain_store(0)
    drain_store(1)


def kernel(x, token_table, pos_table):
    tblT = token_table.T
    wide = _detr_kernel(tblT)
    tok_lin = wide.reshape(VOCAB, D)
    xT = x.astype(jnp.int32).T
    pos_f = pos_table.reshape(L * D)
    out1d = _gather_kernel(xT, tok_lin, pos_f)
    out5d = out1d.reshape(L, D // 8, B // 128, 8, 128)
    return out5d.transpose((2, 4, 0, 1, 3)).reshape(B, L, D)


# speedup vs baseline: 1.7956x; 1.4681x over previous
import functools

import jax
import jax.numpy as jnp
from jax import lax
from jax.experimental import pallas as pl
from jax.experimental.pallas import tpu as pltpu
from jax.experimental.pallas import tpu_sc as plsc

VOCAB = 1000000
L = 200
D = 32
B = 4096
N = B * L

NC, NS = 2, 16
NW = NC * NS

TG = VOCAB // 128
TG_MAIN = (TG // (2 * NW)) * (2 * NW)
K1_PAIRS = TG_MAIN // (2 * NW)


def _iota16():
    return lax.iota(jnp.int32, 16)


@functools.partial(
    pl.kernel,
    mesh=plsc.VectorSubcoreMesh(core_axis_name="c", subcore_axis_name="s"),
    out_type=jax.ShapeDtypeStruct((VOCAB * D,), jnp.float32),
    scratch_types=[
        pltpu.VMEM((32, 128), jnp.float32),
        pltpu.VMEM((32, 128), jnp.float32),
        pltpu.VMEM((4096,), jnp.float32),
        pltpu.VMEM((4096,), jnp.float32),
        pltpu.SemaphoreType.DMA,
        pltpu.SemaphoreType.DMA,
        pltpu.SemaphoreType.DMA,
        pltpu.SemaphoreType.DMA,
    ],
    compiler_params=pltpu.CompilerParams(needs_layout_passes=False),
)
def _detr_kernel(tblT_hbm, out_hbm, stage0, stage1, line0, line1,
                 gsem0, gsem1, ssem0, ssem1):
    stage_v = (stage0, stage1)
    line_v = (line0, line1)
    wid = lax.axis_index("s") * NC + lax.axis_index("c")
    iota = _iota16()
    patq = ((iota >> 2) << 7) + ((iota & 3) << 5)

    def fire_load(g, buf, sem):
        pltpu.async_copy(tblT_hbm.at[:, pl.ds(g * 128, 128)], stage_v[buf], sem)

    def drain_load(buf, sem):
        pltpu.make_async_copy(
            tblT_hbm.at[:, pl.ds(0, 128)], stage_v[buf], sem).wait()

    def drain_store(buf, sem):
        pltpu.make_async_copy(
            line_v[buf], out_hbm.at[pl.ds(0, 4096)], sem).wait()

    def transpose_group(buf, ngroups=8):
        @plsc.parallel_loop(0, 32, unroll=4)
        def d_body(d):
            for tg in range(ngroups):
                v = stage_v[buf][d, pl.ds(tg * 16, 16)]
                plsc.store_scatter(line_v[buf], [patq + (tg * 512 + d)], v)

    def fire_store(g, buf, sem):
        pltpu.async_copy(line_v[buf], out_hbm.at[pl.ds(g * 4096, 4096)], sem)

    fire_load(wid, 0, gsem0)

    def pair_body(i, carry):
        ga = (2 * i) * NW + wid
        gb = (2 * i + 1) * NW + wid
        drain_load(0, gsem0)
        fire_load(gb, 1, gsem1)

        @pl.when(i > 0)
        def _():
            drain_store(0, ssem0)

        transpose_group(0)
        fire_store(ga, 0, ssem0)
        drain_load(1, gsem1)

        @pl.when(i + 1 < K1_PAIRS)
        def _():
            fire_load((2 * i + 2) * NW + wid, 0, gsem0)

        @pl.when(i > 0)
        def _():
            drain_store(1, ssem1)

        transpose_group(1)
        fire_store(gb, 1, ssem1)
        return carry

    lax.fori_loop(0, K1_PAIRS, pair_body, 0)
    drain_store(0, ssem0)
    drain_store(1, ssem1)

    @pl.when(wid < TG - TG_MAIN)
    def _():
        g = TG_MAIN + wid
        fire_load(g, 0, gsem0)
        drain_load(0, gsem0)
        transpose_group(0)
        fire_store(g, 0, ssem0)
        drain_store(0, ssem0)

    @pl.when(wid == 4)
    def _():
        tail_col = wid * (TG * 128 // 4)
        pltpu.async_copy(
            tblT_hbm.at[:, pl.ds(tail_col, 128)], stage_v[1], gsem1)
        drain_load(1, gsem1)
        transpose_group(1, ngroups=4)
        pltpu.async_copy(
            line_v[1].at[pl.ds(0, 2048)],
            out_hbm.at[pl.ds(TG * 4096, 2048)], ssem1)
        pltpu.make_async_copy(
            line_v[1].at[pl.ds(0, 2048)],
            out_hbm.at[pl.ds(0, 2048)], ssem1).wait()


LC = 5
NCHUNK = L // LC
K2_PAIRS = NCHUNK // 2
CH_OUT = LC * D * 128


@functools.partial(
    pl.kernel,
    mesh=plsc.VectorSubcoreMesh(core_axis_name="c", subcore_axis_name="s"),
    out_type=jax.ShapeDtypeStruct((N * D,), jnp.float32),
    scratch_types=[
        pltpu.VMEM((L, 128), jnp.int32),
        pltpu.VMEM((L * D,), jnp.float32),
        pltpu.VMEM((LC, 128, D), jnp.float32),
        pltpu.VMEM((LC, 128, D), jnp.float32),
        pltpu.VMEM((CH_OUT,), jnp.float32),
        pltpu.VMEM((CH_OUT,), jnp.float32),
        pltpu.SemaphoreType.DMA,
        pltpu.SemaphoreType.DMA,
        pltpu.SemaphoreType.DMA,
        pltpu.SemaphoreType.DMA,
    ],
    compiler_params=pltpu.CompilerParams(
        use_tc_tiling_on_sc=False, needs_layout_passes=False),
)
def _gather_kernel(xT_hbm, tok_hbm, pos_hbm, out_hbm, idx_v, pos_v, rows0,
                   rows1, st0, st1, gsem0, gsem1, ssem0, ssem1):
    rows_v = (rows0, rows1)
    st_v = (st0, st1)
    wid = lax.axis_index("s") * NC + lax.axis_index("c")
    gsem = (gsem0, gsem1)
    ssem = (ssem0, ssem1)
    iota = _iota16()
    patd = ((iota >> 3) << 10) + ((iota & 7) << 7)

    pltpu.sync_copy(xT_hbm.at[:, pl.ds(wid * 128, 128)], idx_v)
    pltpu.sync_copy(pos_hbm, pos_v)

    def fire_gathers(c, buf):
        l0 = c * LC
        for li in range(LC):
            pltpu.async_copy(
                tok_hbm.at[idx_v.at[l0 + li]], rows_v[buf].at[li], gsem[buf])

    def drain_gathers_all(buf):
        for _ in range(LC):
            pltpu.make_async_copy(
                tok_hbm.at[idx_v.at[0]], rows_v[buf].at[0], gsem[buf]).wait()

    def drain_store(buf):
        pltpu.make_async_copy(
            st_v[buf], out_hbm.at[pl.ds(0, CH_OUT)], ssem[buf]).wait()

    def transpose_chunk(c, buf):
        l0 = c * LC
        pv = [[pos_v[pl.ds((l0 + li) * D + h * 16, 16)] for h in range(2)]
              for li in range(LC)]

        @plsc.parallel_loop(0, 128, unroll=2)
        def b_body(b):
            for li in range(LC):
                for h in range(2):
                    v = rows_v[buf][li, b, pl.ds(h * 16, 16)] + pv[li][h]
                    plsc.store_scatter(
                        st_v[buf],
                        [patd + (li * 4096 + h * 2048 + b)], v)

    def fire_store(c, buf):
        l0 = c * LC
        for li in range(LC):
            for dt in range(4):
                pltpu.async_copy(
                    st_v[buf].at[pl.ds((li * 4 + dt) * 1024, 1024)],
                    out_hbm.at[pl.ds(((l0 + li) * 4 + dt) * (B // 128 * 1024)
                                     + wid * 1024, 1024)],
                    ssem[buf])

    fire_gathers(0, 0)

    def pair_body(i, carry):
        ca = 2 * i
        cb = 2 * i + 1
        fire_gathers(cb, 1)
        drain_gathers_all(0)

        @pl.when(i > 0)
        def _():
            drain_store(0)

        transpose_chunk(ca, 0)
        fire_store(ca, 0)

        @pl.when(i + 1 < K2_PAIRS)
        def _():
            fire_gathers(ca + 2, 0)

        drain_gathers_all(1)

        @pl.when(i > 0)
        def _():
            drain_store(1)

        transpose_chunk(cb, 1)
        fire_store(cb, 1)
        return carry

    lax.fori_loop(0, K2_PAIRS, pair_body, 0)
    drain_store(0)
    drain_store(1)


def kernel(x, token_table, pos_table):
    tblT = token_table.T
    wide = _detr_kernel(tblT)
    tok_lin = wide.reshape(VOCAB, D)
    xT = x.astype(jnp.int32).T
    pos_f = pos_table.reshape(L * D)
    out1d = _gather_kernel(xT, tok_lin, pos_f)
    out5d = out1d.reshape(L, D // 8, B // 128, 8, 128)
    return out5d.transpose((2, 4, 0, 1, 3)).reshape(B, L, D)
